# baseline (speedup 1.0000x reference)
import functools

import jax
import jax.numpy as jnp
import numpy as np
from jax import lax
from jax.experimental import pallas as pl
from jax.experimental.pallas import tpu as pltpu
from jax.experimental.pallas import tpu_sc as plsc

_TEMP = np.float32(0.7)
_INV_TEMP = np.float32(1.0) / _TEMP


def _tc_manual(h, w, t_off, T_TC, D, E, RB, NBUF):
    n_blk = T_TC // RB
    assert n_blk % NBUF == 0

    def body(h_hbm, w_ref, o_ref, bufs, sems):
        def start(b, s):
            pltpu.make_async_copy(
                h_hbm.at[pl.ds(t_off + b * RB, RB)], bufs.at[s], sems.at[s]
            ).start()

        def wait(s):
            pltpu.make_async_copy(
                h_hbm.at[pl.ds(t_off, RB)], bufs.at[s], sems.at[s]
            ).wait()

        for s in range(NBUF):
            start(s, s)
        w = w_ref[...]

        def grp_body(g, carry):
            b0 = g * NBUF
            for s in range(NBUF):
                wait(s)
                r = lax.dot_general(
                    w, bufs[s],
                    (((1,), (1,)), ((), ())),
                    preferred_element_type=jnp.float32,
                )
                o_ref[:, pl.ds((b0 + s) * RB, RB)] = r * _INV_TEMP

                @pl.when(b0 + s + NBUF < n_blk)
                def _():
                    start(b0 + s + NBUF, s)

            return carry

        lax.fori_loop(0, n_blk // NBUF, grp_body, 0)

    return pl.pallas_call(
        body,
        in_specs=[
            pl.BlockSpec(memory_space=pl.ANY),
            pl.BlockSpec((E, D), lambda: (0, 0)),
        ],
        out_specs=pl.BlockSpec((E, T_TC), lambda: (0, 0)),
        out_shape=jax.ShapeDtypeStruct((E, T_TC), jnp.float32),
        scratch_shapes=[
            pltpu.VMEM((NBUF, RB, D), jnp.float32),
            pltpu.SemaphoreType.DMA((NBUF,)),
        ],
    )(h, w)


def kernel(hidden_states, gate_weight):
    B, S, D = hidden_states.shape
    E = gate_weight.shape[0]
    T = B * S
    h = hidden_states.reshape(T, D)
    out = _tc_manual(h, gate_weight, 0, T, D, E, RB=256, NBUF=8)
    return out.T.reshape(B, S, E)
